# Initial kernel scaffold; baseline (speedup 1.0000x reference)
import functools

import jax
import jax.numpy as jnp
from jax import lax
from jax.experimental import pallas as pl
from jax.experimental.pallas import tpu as pltpu
from jax.experimental.pallas import tpu_sc as plsc

_CHUNK = 128


@functools.partial(jax.jit, static_argnames=("nw", "nchunk"))
def _embed(x3, table, *, nw, nchunk):
    D = table.shape[1]
    b_per_w = nchunk * _CHUNK
    B = nw * b_per_w

    mesh = plsc.VectorSubcoreMesh(core_axis_name="c", subcore_axis_name="s")

    @functools.partial(
        pl.kernel,
        mesh=mesh,
        out_type=jax.ShapeDtypeStruct((B, D), jnp.float32),
        scratch_types=[
            pltpu.VMEM((nchunk, _CHUNK), jnp.int32),
            pltpu.VMEM((_CHUNK, D), jnp.float32),
            pltpu.SemaphoreType.DMA,
        ],
    )
    def emb(table_hbm, x_hbm, out_hbm, idx_v, rows_v, sem):
        wid = lax.axis_index("s") * 2 + lax.axis_index("c")
        base = wid * b_per_w
        pltpu.sync_copy(x_hbm.at[wid], idx_v)

        def chunk(j, carry):
            pltpu.async_copy(table_hbm.at[idx_v.at[j]], rows_v, sem).wait()
            pltpu.sync_copy(rows_v, out_hbm.at[pl.ds(base + j * _CHUNK, _CHUNK)])
            return carry

        lax.fori_loop(0, nchunk, chunk, 0)

    return emb(table, x3)


def kernel(x, table):
    orig_shape = x.shape
    D = table.shape[1]
    B = x.size
    nw = 32
    b_per_w = B // nw
    nchunk = b_per_w // _CHUNK
    x3 = x.reshape(nw, nchunk, _CHUNK).astype(jnp.int32)
    out = _embed(x3, table, nw=nw, nchunk=nchunk)
    return out.reshape(*orig_shape, D)


# speedup vs baseline: 1.6832x; 1.6832x over previous
import functools

import jax
import jax.numpy as jnp
from jax import lax
from jax.experimental import pallas as pl
from jax.experimental.pallas import tpu as pltpu
from jax.experimental.pallas import tpu_sc as plsc

_CHUNK = 128


@functools.partial(jax.jit, static_argnames=("nw", "nchunk"))
def _embed(x3, table, *, nw, nchunk):
    D = table.shape[1]
    b_per_w = nchunk * _CHUNK
    B = nw * b_per_w

    mesh = plsc.VectorSubcoreMesh(core_axis_name="c", subcore_axis_name="s")

    @functools.partial(
        pl.kernel,
        mesh=mesh,
        out_type=jax.ShapeDtypeStruct((B, D), jnp.float32),
        compiler_params=pltpu.CompilerParams(use_tc_tiling_on_sc=False),
        scratch_types=[
            pltpu.VMEM((nchunk, _CHUNK), jnp.int32),
            pltpu.VMEM((_CHUNK, D), jnp.float32),
            pltpu.SemaphoreType.DMA,
        ],
    )
    def emb(table_hbm, x_hbm, out_hbm, idx_v, rows_v, sem):
        wid = lax.axis_index("s") * 2 + lax.axis_index("c")
        base = wid * b_per_w
        pltpu.sync_copy(x_hbm.at[wid], idx_v)

        def chunk(j, carry):
            pltpu.async_copy(table_hbm.at[idx_v.at[j]], rows_v, sem).wait()
            pltpu.sync_copy(rows_v, out_hbm.at[pl.ds(base + j * _CHUNK, _CHUNK)])
            return carry

        lax.fori_loop(0, nchunk, chunk, 0)

    return emb(table, x3)


def kernel(x, table):
    orig_shape = x.shape
    D = table.shape[1]
    B = x.size
    nw = 32
    b_per_w = B // nw
    nchunk = b_per_w // _CHUNK
    x3 = x.reshape(nw, nchunk, _CHUNK).astype(jnp.int32)
    out = _embed(x3, table, nw=nw, nchunk=nchunk)
    return out.reshape(*orig_shape, D)


# speedup vs baseline: 1.8748x; 1.1139x over previous
import functools

import jax
import jax.numpy as jnp
from jax import lax
from jax.experimental import pallas as pl
from jax.experimental.pallas import tpu as pltpu
from jax.experimental.pallas import tpu_sc as plsc

_CHUNK = 128
_NBUF = 8
_FIRE = 4


@functools.partial(jax.jit, static_argnames=("nw", "nchunk"))
def _embed(x3, table, *, nw, nchunk):
    D = table.shape[1]
    b_per_w = nchunk * _CHUNK
    B = nw * b_per_w
    ngroups = nchunk // _NBUF
    assert nchunk % _NBUF == 0 and ngroups >= 2

    mesh = plsc.VectorSubcoreMesh(core_axis_name="c", subcore_axis_name="s")

    @functools.partial(
        pl.kernel,
        mesh=mesh,
        out_type=jax.ShapeDtypeStruct((B, D), jnp.float32),
        compiler_params=pltpu.CompilerParams(use_tc_tiling_on_sc=False),
        scratch_types=(
            [pltpu.VMEM((nchunk, _CHUNK), jnp.int32),
             pltpu.VMEM((_NBUF, _CHUNK, D), jnp.float32)]
            + [pltpu.SemaphoreType.DMA] * (2 * _NBUF)
        ),
    )
    def emb(table_hbm, x_hbm, out_hbm, idx_v, rows_v, *sems):
        gsem = sems[:_NBUF]
        osem = sems[_NBUF:]
        wid = lax.axis_index("s") * 2 + lax.axis_index("c")
        base = wid * b_per_w
        pltpu.sync_copy(x_hbm.at[wid], idx_v)

        def fire_gather(c, slot):
            pltpu.async_copy(table_hbm.at[idx_v.at[c]], rows_v.at[slot],
                             gsem[slot])

        def wait_gather(c, slot):
            pltpu.make_async_copy(table_hbm.at[idx_v.at[c]], rows_v.at[slot],
                                  gsem[slot]).wait()

        def fire_out(j, slot):
            pltpu.async_copy(rows_v.at[slot],
                             out_hbm.at[pl.ds(base + j * _CHUNK, _CHUNK)],
                             osem[slot])

        def wait_out(j, slot):
            pltpu.make_async_copy(rows_v.at[slot],
                                  out_hbm.at[pl.ds(base + j * _CHUNK, _CHUNK)],
                                  osem[slot]).wait()

        def step(j, k):
            c = j + _FIRE
            kc = (k + _FIRE) % _NBUF
            wait_out(c - _NBUF, kc)
            fire_gather(c, kc)
            wait_gather(j, k)
            fire_out(j, k)

        for c in range(_FIRE):
            fire_gather(c, c)
        for k in range(_NBUF):
            j = k
            c = j + _FIRE
            kc = (k + _FIRE) % _NBUF
            if c >= _NBUF:
                wait_out(c - _NBUF, kc)
            fire_gather(c, kc)
            wait_gather(j, k)
            fire_out(j, k)

        def group(g, carry):
            for k in range(_NBUF):
                step(g * _NBUF + k, k)
            return carry

        lax.fori_loop(1, ngroups - 1, group, 0)

        for k in range(_NBUF):
            j = (ngroups - 1) * _NBUF + k
            c = j + _FIRE
            if c < nchunk:
                kc = (k + _FIRE) % _NBUF
                wait_out(c - _NBUF, kc)
                fire_gather(c, kc)
            wait_gather(j, k)
            fire_out(j, k)
        for k in range(_NBUF):
            wait_out((ngroups - 1) * _NBUF + k, k)

    return emb(table, x3)


def kernel(x, table):
    orig_shape = x.shape
    D = table.shape[1]
    B = x.size
    nw = 32
    b_per_w = B // nw
    nchunk = b_per_w // _CHUNK
    x3 = x.reshape(nw, nchunk, _CHUNK).astype(jnp.int32)
    out = _embed(x3, table, nw=nw, nchunk=nchunk)
    return out.reshape(*orig_shape, D)
